# baseline (speedup 1.0000x reference)
import functools

import jax
import jax.numpy as jnp
from jax import lax
from jax.experimental import pallas as pl
from jax.experimental.pallas import tpu as pltpu
from jax.experimental.pallas import tpu_sc as plsc

VOCAB = 1000000
NUM_CORES = 2
NUM_SUBCORES = 16
NUM_WORKERS = NUM_CORES * NUM_SUBCORES
LANES = 16
CHUNK = 128
CPIECE = 1600


def _stay_body(th_ref, tn_ref, dh_ref, dn_ref, out_ref):
    cond = dh_ref[...] != dn_ref[...]
    out_ref[...] = jnp.where(cond, 48, 0) + tn_ref[...] - th_ref[...]


def _stay_time(ts_his, ts_next, day_his, day_next):
    n_rows, n_cols = ts_his.shape
    block = 512
    grid = n_rows // block
    spec = pl.BlockSpec((block, n_cols), lambda i: (i, 0))
    return pl.pallas_call(
        _stay_body,
        grid=(grid,),
        in_specs=[spec, spec, spec, spec],
        out_specs=spec,
        out_shape=jax.ShapeDtypeStruct((n_rows, n_cols), jnp.int32),
    )(ts_his, ts_next, day_his, day_next)


def _make_gather(n_idx, embed_dim):
    per_w = n_idx // NUM_WORKERS
    n_chunks = per_w // CHUNK

    @functools.partial(
        pl.kernel,
        out_type=jax.ShapeDtypeStruct((n_idx, embed_dim), jnp.float32),
        mesh=plsc.VectorSubcoreMesh(
            core_axis_name="c",
            subcore_axis_name="s",
            num_cores=NUM_CORES,
            num_subcores=NUM_SUBCORES,
        ),
        scratch_types=[
            pltpu.VMEM((per_w,), jnp.int32),
            pltpu.VMEM((CPIECE,), jnp.int32),
            pltpu.VMEM((4, CHUNK, 128), jnp.float32),
            pltpu.VMEM((CHUNK, 64), jnp.float32),
            pltpu.VMEM((CHUNK, 64), jnp.float32),
            pltpu.SemaphoreType.DMA,
            pltpu.SemaphoreType.DMA,
            pltpu.SemaphoreType.DMA,
            pltpu.SemaphoreType.DMA,
            pltpu.SemaphoreType.DMA,
            pltpu.SemaphoreType.DMA,
        ],
    )
    def gather_kernel(tab_hbm, x_hbm, city_hbm, out_hbm, idx_v, city_v,
                      ring, cmp0, cmp1, g0, g1, g2, g3, ssem0, ssem1):
        wid = lax.axis_index("c") * NUM_SUBCORES + lax.axis_index("s")
        wbase = wid * per_w
        gsems = (g0, g1, g2, g3)
        cmps = (cmp0, cmp1)
        ssems = (ssem0, ssem1)
        pltpu.sync_copy(x_hbm.at[pl.ds(wbase, per_w)], idx_v)

        def piece_body(p, carry):
            pltpu.sync_copy(
                city_hbm.at[pl.ds(wbase + p * CPIECE, CPIECE)], city_v
            )

            def idx_body(j, carry2):
                o = j * LANES
                s = idx_v[pl.ds(p * CPIECE + o, LANES)] + city_v[pl.ds(o, LANES)]
                idx_v[pl.ds(p * CPIECE + o, LANES)] = jnp.where(
                    s >= VOCAB, s - VOCAB, s
                )
                return carry2

            lax.fori_loop(0, CPIECE // LANES, idx_body, 0)
            return carry

        lax.fori_loop(0, per_w // CPIECE, piece_body, 0)

        def start_gather(ci, m):
            return pltpu.async_copy(
                tab_hbm.at[idx_v.at[pl.ds(ci * CHUNK, CHUNK)]],
                ring.at[m], gsems[m]
            )

        def wait_gather(m):
            pltpu.make_async_copy(
                tab_hbm.at[idx_v.at[pl.ds(0, CHUNK)]], ring.at[m], gsems[m]
            ).wait()

        def compact(m, cmp):
            def row_body(r, carry2):
                for j in range(64 // LANES):
                    cmp[r, pl.ds(j * LANES, LANES)] = (
                        ring[m, r, pl.ds(j * LANES, LANES)]
                    )
                return carry2

            lax.fori_loop(0, CHUNK, row_body, 0)

        def start_out(ci, cmp, ssem):
            return pltpu.async_copy(
                cmp, out_hbm.at[pl.ds(wbase + ci * CHUNK, CHUNK)], ssem
            )

        def wait_out(cmp, ssem):
            pltpu.make_async_copy(
                cmp, out_hbm.at[pl.ds(wbase, CHUNK)], ssem
            ).wait()

        start_gather(0, 0)
        start_gather(1, 1)
        start_gather(2, 2)

        def quad_body(k, carry):
            c0 = 4 * k
            for m in range(4):
                c = c0 + m

                @pl.when(c + 3 < n_chunks)
                def _():
                    start_gather(c + 3, (m + 3) % 4)

                wait_gather(m)
                cmp = cmps[m % 2]
                ssem = ssems[m % 2]

                @pl.when(c >= 2)
                def _():
                    wait_out(cmp, ssem)

                compact(m, cmp)
                start_out(c, cmp, ssem)
            return carry

        lax.fori_loop(0, n_chunks // 4, quad_body, 0)
        wait_out(cmp0, ssem0)
        wait_out(cmp1, ssem1)

    return gather_kernel


def kernel(traj, vocab_tot):
    batch, hist_p1, _ = traj.shape
    his_len = hist_p1 - 1
    t = traj.astype(jnp.int32)
    x_test = t[:, :-1, 0]
    y_test = t[:, 1:, 0]
    ts_his = t[:, :-1, 1]
    ts_next = t[:, 1:, 1]
    day_his = t[:, :-1, 2]
    day_next = t[:, 1:, 2]
    test_city = t[:, :-1, 3]

    stay_time = _stay_time(ts_his, ts_next, day_his, day_next)

    n_idx = batch * his_len
    vocab_rows, embed_dim = vocab_tot.shape
    tabpad = jnp.pad(vocab_tot, ((0, 0), (0, 128 - embed_dim)))

    gather = _make_gather(n_idx, embed_dim)
    poi = gather(tabpad, x_test.reshape(n_idx), test_city.reshape(n_idx))
    return poi.reshape(batch, his_len, embed_dim), stay_time, y_test


# speedup vs baseline: 1.0116x; 1.0116x over previous
import functools

import jax
import jax.numpy as jnp
from jax import lax
from jax.experimental import pallas as pl
from jax.experimental.pallas import tpu as pltpu
from jax.experimental.pallas import tpu_sc as plsc

VOCAB = 1000000
NUM_CORES = 2
NUM_SUBCORES = 16
NUM_WORKERS = NUM_CORES * NUM_SUBCORES
LANES = 16
CHUNK = 128


def _stay_body(th_ref, tn_ref, dh_ref, dn_ref, out_ref):
    cond = dh_ref[...] != dn_ref[...]
    out_ref[...] = jnp.where(cond, 48, 0) + tn_ref[...] - th_ref[...]


def _stay_time(ts_his, ts_next, day_his, day_next):
    n_rows, n_cols = ts_his.shape
    block = 512
    grid = n_rows // block
    spec = pl.BlockSpec((block, n_cols), lambda i: (i, 0))
    return pl.pallas_call(
        _stay_body,
        grid=(grid,),
        in_specs=[spec, spec, spec, spec],
        out_specs=spec,
        out_shape=jax.ShapeDtypeStruct((n_rows, n_cols), jnp.int32),
    )(ts_his, ts_next, day_his, day_next)


def _make_gather(n_idx, embed_dim):
    per_w = n_idx // NUM_WORKERS
    n_chunks = per_w // CHUNK

    @functools.partial(
        pl.kernel,
        out_type=jax.ShapeDtypeStruct((n_idx, embed_dim), jnp.float32),
        mesh=plsc.VectorSubcoreMesh(
            core_axis_name="c",
            subcore_axis_name="s",
            num_cores=NUM_CORES,
            num_subcores=NUM_SUBCORES,
        ),
        scratch_types=[
            pltpu.VMEM((per_w,), jnp.int32),
            pltpu.VMEM((per_w,), jnp.int32),
            pltpu.VMEM((CHUNK, 128), jnp.float32),
            pltpu.VMEM((CHUNK, 128), jnp.float32),
            pltpu.VMEM((CHUNK, 64), jnp.float32),
            pltpu.VMEM((CHUNK, 64), jnp.float32),
            pltpu.SemaphoreType.DMA,
            pltpu.SemaphoreType.DMA,
            pltpu.SemaphoreType.DMA,
            pltpu.SemaphoreType.DMA,
        ],
    )
    def gather_kernel(tab_hbm, x_hbm, city_hbm, out_hbm, idx_v, city_v,
                      rows0, rows1, cmp0, cmp1, gsem0, gsem1, ssem0, ssem1):
        wid = lax.axis_index("c") * NUM_SUBCORES + lax.axis_index("s")
        wbase = wid * per_w
        pltpu.sync_copy(x_hbm.at[pl.ds(wbase, per_w)], idx_v)
        pltpu.sync_copy(city_hbm.at[pl.ds(wbase, per_w)], city_v)

        def idx_body(j, carry):
            o = j * LANES
            s = idx_v[pl.ds(o, LANES)] + city_v[pl.ds(o, LANES)]
            idx_v[pl.ds(o, LANES)] = jnp.where(s >= VOCAB, s - VOCAB, s)
            return carry

        lax.fori_loop(0, per_w // LANES, idx_body, 0)

        def start_gather(ci, rows, gsem):
            return pltpu.async_copy(
                tab_hbm.at[idx_v.at[pl.ds(ci * CHUNK, CHUNK)]], rows, gsem
            )

        def compact(rows, cmp):
            def row_body(r, carry2):
                for j in range(64 // LANES):
                    cmp[r, pl.ds(j * LANES, LANES)] = (
                        rows[r, pl.ds(j * LANES, LANES)]
                    )
                return carry2

            lax.fori_loop(0, CHUNK, row_body, 0)

        def start_out(ci, cmp, ssem):
            return pltpu.async_copy(
                cmp, out_hbm.at[pl.ds(wbase + ci * CHUNK, CHUNK)], ssem
            )

        start_gather(0, rows0, gsem0)

        def pair_body(k, carry):
            c0 = 2 * k
            start_gather(c0 + 1, rows1, gsem1)
            pltpu.make_async_copy(tab_hbm.at[idx_v.at[pl.ds(0, CHUNK)]],
                                  rows0, gsem0).wait()

            @pl.when(k > 0)
            def _():
                pltpu.make_async_copy(
                    cmp0, out_hbm.at[pl.ds(wbase, CHUNK)], ssem0
                ).wait()

            compact(rows0, cmp0)
            start_out(c0, cmp0, ssem0)

            @pl.when(c0 + 2 < n_chunks)
            def _():
                start_gather(c0 + 2, rows0, gsem0)

            pltpu.make_async_copy(tab_hbm.at[idx_v.at[pl.ds(0, CHUNK)]],
                                  rows1, gsem1).wait()

            @pl.when(k > 0)
            def _():
                pltpu.make_async_copy(
                    cmp1, out_hbm.at[pl.ds(wbase, CHUNK)], ssem1
                ).wait()

            compact(rows1, cmp1)
            start_out(c0 + 1, cmp1, ssem1)
            return carry

        lax.fori_loop(0, n_chunks // 2, pair_body, 0)
        pltpu.make_async_copy(cmp0, out_hbm.at[pl.ds(wbase, CHUNK)],
                              ssem0).wait()
        pltpu.make_async_copy(cmp1, out_hbm.at[pl.ds(wbase, CHUNK)],
                              ssem1).wait()

    return gather_kernel


def kernel(traj, vocab_tot):
    batch, hist_p1, _ = traj.shape
    his_len = hist_p1 - 1
    t = traj.astype(jnp.int32)
    x_test = t[:, :-1, 0]
    y_test = t[:, 1:, 0]
    ts_his = t[:, :-1, 1]
    ts_next = t[:, 1:, 1]
    day_his = t[:, :-1, 2]
    day_next = t[:, 1:, 2]
    test_city = t[:, :-1, 3]

    stay_time = _stay_time(ts_his, ts_next, day_his, day_next)

    n_idx = batch * his_len
    vocab_rows, embed_dim = vocab_tot.shape
    tabpad = jnp.pad(vocab_tot, ((0, 0), (0, 128 - embed_dim)))

    gather = _make_gather(n_idx, embed_dim)
    poi = gather(tabpad, x_test.reshape(n_idx), test_city.reshape(n_idx))
    return poi.reshape(batch, his_len, embed_dim), stay_time, y_test
